# baseline (speedup 1.0000x reference)
import functools

import jax
import jax.numpy as jnp
from jax import lax
from jax.experimental import pallas as pl
from jax.experimental.pallas import tpu as pltpu
from jax.experimental.pallas import tpu_sc as plsc

N = 10000
E = 320000
EL = E + N
IN_DIM = 128
HID = 64
OUT_DIM = 32

NC = 2
NS = 16
NW = NC * NS
CH = 128
NCHT = 162
EPAD = NS * NCHT * CH
NCHD = EPAD // (NW * CH)
NP = 10240
RPT = NP // NS
DUMP = N
K = 6
NG = NCHT // K

RB = 2048


def _sc_mesh():
    return plsc.VectorSubcoreMesh(
        core_axis_name="c", subcore_axis_name="s",
        num_cores=NC, num_subcores=NS)


@functools.lru_cache(maxsize=None)
def _make_agg(dh, col_split, ncht, k):
    dhi = dh // 2
    ng = ncht // k

    @functools.partial(
        pl.kernel,
        out_type=pltpu.HBM((NC, NP, dh), jnp.float32),
        mesh=_sc_mesh(),
        compiler_params=pltpu.CompilerParams(
            use_tc_tiling_on_sc=False, needs_layout_passes=False),
        scratch_types=(
            [pltpu.VMEM((ncht, CH), jnp.int32)] * 2
            + [pltpu.VMEM((CH, dhi), jnp.int32)] * k
            + [pltpu.VMEM((CH, dh), jnp.float32)] * (k + 1)
            + [pltpu.SemaphoreType.DMA] * (2 * k)
            + [pltpu.VMEM_SHARED((NP, dhi), jnp.int32),
               pltpu.VMEM_SHARED((NP, dh), jnp.float32)]
        ),
    )
    def agg_kernel(y_hbm, src_hbm, dst_hbm, out_hbm, src_idx, dst_idx, *rest):
        ibufs = rest[:k]
        fbufs = rest[k:2 * k]
        zb = rest[2 * k]
        gsem = rest[2 * k + 1:3 * k + 1]
        ssem = rest[3 * k + 1:4 * k + 1]
        y_sh = rest[4 * k + 1]
        acc = rest[4 * k + 2]
        c = lax.axis_index("c")
        s = lax.axis_index("s")
        if col_split:
            row0 = s * ncht
            y_rows = lambda r, n: y_hbm.at[c, pl.ds(r, n)]
        else:
            row0 = (s * NC + c) * ncht
            y_rows = lambda r, n: y_hbm.at[pl.ds(r, n)]
        pltpu.sync_copy(src_hbm.at[pl.ds(row0, ncht)], src_idx)
        pltpu.sync_copy(dst_hbm.at[pl.ds(row0, ncht)], dst_idx)
        nq = RPT // CH
        for q in range(2):
            pltpu.async_copy(y_rows(s * RPT + q * CH, CH), ibufs[q], gsem[q])
        for q in range(nq):
            pltpu.make_async_copy(y_rows(s * RPT + q * CH, CH),
                                  ibufs[q % 2], gsem[q % 2]).wait()
            pltpu.sync_copy(ibufs[q % 2], y_sh.at[pl.ds(s * RPT + q * CH, CH)])
            if q + 2 < nq:
                pltpu.async_copy(y_rows(s * RPT + (q + 2) * CH, CH),
                                 ibufs[q % 2], gsem[q % 2])
        zeros16 = jnp.zeros((16,), jnp.float32)

        @pl.loop(0, CH)
        def zrow(i):
            for kk in range(dh // 16):
                zb[i, pl.ds(kk * 16, 16)] = zeros16

        for q in range(nq):
            pltpu.sync_copy(zb, acc.at[pl.ds(s * RPT + q * CH, CH)])
        plsc.subcore_barrier()

        mask_hi = jnp.full((16,), -65536, jnp.int32)

        def convert(ib, fb):
            @pl.loop(0, CH, unroll=8)
            def crow(r):
                for kk in range(dhi // 16):
                    v = ib[r, pl.ds(16 * kk, 16)]
                    fb[r, pl.ds(16 * kk, 16)] = plsc.bitcast(
                        v << 16, jnp.float32)
                    fb[r, pl.ds(dhi + 16 * kk, 16)] = plsc.bitcast(
                        v & mask_hi, jnp.float32)

        for b in range(k):
            pltpu.async_copy(y_sh.at[src_idx.at[b]], ibufs[b], gsem[b])

        @pl.loop(0, ng - 1)
        def group(g):
            base = g * k
            for b in range(k):
                pltpu.make_async_copy(
                    y_sh.at[src_idx.at[base + b]], ibufs[b], gsem[b]).wait()
                convert(ibufs[b], fbufs[b])
                pltpu.async_copy(fbufs[b], acc.at[dst_idx.at[base + b]],
                                 ssem[b], add=True)
            for b in range(k):
                pltpu.make_async_copy(
                    fbufs[b], acc.at[dst_idx.at[base + b]], ssem[b]).wait()
                pltpu.async_copy(
                    y_sh.at[src_idx.at[base + k + b]], ibufs[b], gsem[b])

        base = (ng - 1) * k
        for b in range(k):
            pltpu.make_async_copy(
                y_sh.at[src_idx.at[base + b]], ibufs[b], gsem[b]).wait()
            convert(ibufs[b], fbufs[b])
            pltpu.async_copy(fbufs[b], acc.at[dst_idx.at[base + b]],
                             ssem[b], add=True)
        for b in range(k):
            pltpu.make_async_copy(
                fbufs[b], acc.at[dst_idx.at[base + b]], ssem[b]).wait()
        plsc.subcore_barrier()
        pltpu.sync_copy(acc.at[pl.ds(s * RPT, RPT)],
                        out_hbm.at[c, pl.ds(s * RPT, RPT)])

    return agg_kernel


@functools.lru_cache(maxsize=None)
def _make_deg():
    @functools.partial(
        pl.kernel,
        out_type=pltpu.HBM((NC, NP), jnp.float32),
        mesh=_sc_mesh(),
        compiler_params=pltpu.CompilerParams(use_tc_tiling_on_sc=False),
        scratch_types=(
            pltpu.VMEM((NCHD, CH), jnp.int32),
            pltpu.VMEM((CH,), jnp.float32),
            pltpu.VMEM_SHARED((NP,), jnp.float32),
        ),
    )
    def deg_kernel(dst_hbm, ones_hbm, zrow_hbm, out_hbm, dst_idx, ones_v, acc):
        c = lax.axis_index("c")
        s = lax.axis_index("s")
        wid = s * NC + c
        pltpu.sync_copy(zrow_hbm, acc.at[pl.ds(s * RPT, RPT)])
        pltpu.sync_copy(ones_hbm, ones_v)
        pltpu.sync_copy(dst_hbm.at[pl.ds(wid * NCHD, NCHD)], dst_idx)
        plsc.subcore_barrier()

        @pl.loop(0, NCHD)
        def chunk(j):
            pltpu.sync_copy(ones_v, acc.at[dst_idx.at[j]], add=True)

        plsc.subcore_barrier()
        pltpu.sync_copy(acc.at[pl.ds(s * RPT, RPT)],
                        out_hbm.at[c, pl.ds(s * RPT, RPT)])

    return deg_kernel


def _pack_bf16(lo, hi):
    lo32 = lax.bitcast_convert_type(
        lo.astype(jnp.bfloat16), jnp.uint16).astype(jnp.int32)
    hi32 = lax.bitcast_convert_type(
        hi.astype(jnp.bfloat16), jnp.uint16).astype(jnp.int32)
    return jnp.bitwise_or(lo32, jnp.left_shift(hi32, 16))


def _tc_layer1(xp, degp, W1):
    H2 = HID // 2
    H4 = HID // 4

    def body(x_ref, deg_ref, w_ref, y_ref, dinv_ref):
        deg = deg_ref[0, :] + deg_ref[1, :]
        s = lax.rsqrt(deg)
        y = jnp.dot(x_ref[...], w_ref[...],
                    preferred_element_type=jnp.float32) * s[:, None]
        y_ref[0, :, :] = _pack_bf16(y[:, 0:H4], y[:, H4:H2])
        y_ref[1, :, :] = _pack_bf16(y[:, H2:H2 + H4], y[:, H2 + H4:])
        dinv_ref[0, 0, :] = s

    return pl.pallas_call(
        body,
        grid=(NP // RB,),
        in_specs=[
            pl.BlockSpec((RB, IN_DIM), lambda i: (i, 0)),
            pl.BlockSpec((NC, RB), lambda i: (0, i)),
            pl.BlockSpec((IN_DIM, HID), lambda i: (0, 0)),
        ],
        out_specs=[
            pl.BlockSpec((NC, RB, H4), lambda i: (0, i, 0)),
            pl.BlockSpec((1, 1, RB), lambda i: (i, 0, 0)),
        ],
        out_shape=[
            jax.ShapeDtypeStruct((NC, NP, H4), jnp.int32),
            jax.ShapeDtypeStruct((NP // RB, 1, RB), jnp.float32),
        ],
    )(xp, degp, W1)


def _tc_mid(agg1, dinv, b1, W2):
    H2 = HID // 2
    O2 = OUT_DIM // 2

    def body(a_ref, dinv_ref, b_ref, w_ref, y2_ref):
        s = dinv_ref[0, 0, :]
        b = b_ref[...]
        w = w_ref[...]
        hl = jnp.maximum(a_ref[0] * s[:, None] + b[:H2], 0.0)
        hr = jnp.maximum(a_ref[1] * s[:, None] + b[H2:], 0.0)
        y2 = (jnp.dot(hl, w[:H2, :], preferred_element_type=jnp.float32)
              + jnp.dot(hr, w[H2:, :], preferred_element_type=jnp.float32))
        y2 = y2 * s[:, None]
        y2_ref[...] = _pack_bf16(y2[:, :O2], y2[:, O2:])

    return pl.pallas_call(
        body,
        grid=(NP // RB,),
        in_specs=[
            pl.BlockSpec((NC, RB, H2), lambda i: (0, i, 0)),
            pl.BlockSpec((1, 1, RB), lambda i: (i, 0, 0)),
            pl.BlockSpec((HID,), lambda i: (0,)),
            pl.BlockSpec((HID, OUT_DIM), lambda i: (0, 0)),
        ],
        out_specs=pl.BlockSpec((RB, O2), lambda i: (i, 0)),
        out_shape=jax.ShapeDtypeStruct((NP, O2), jnp.int32),
    )(agg1, dinv, b1, W2)


def _tc_out(agg2, dinv, b2):
    def body(a_ref, dinv_ref, b_ref, o_ref):
        s = dinv_ref[0, 0, :]
        o = (a_ref[0] + a_ref[1]) * s[:, None] + b_ref[...]
        o_ref[...] = jnp.maximum(o, 0.0)

    return pl.pallas_call(
        body,
        grid=(NP // RB,),
        in_specs=[
            pl.BlockSpec((NC, RB, OUT_DIM), lambda i: (0, i, 0)),
            pl.BlockSpec((1, 1, RB), lambda i: (i, 0, 0)),
            pl.BlockSpec((OUT_DIM,), lambda i: (0,)),
        ],
        out_specs=pl.BlockSpec((RB, OUT_DIM), lambda i: (i, 0)),
        out_shape=jax.ShapeDtypeStruct((NP, OUT_DIM), jnp.float32),
    )(agg2, dinv, b2)


def kernel(x, edge_index, W1, b1, W2, b2):
    ei = edge_index.astype(jnp.int32)
    loop = jnp.arange(N, dtype=jnp.int32)
    pad = jnp.full((EPAD - EL,), DUMP, jnp.int32)
    srcp = jnp.concatenate([ei[0], loop, pad]).reshape(NS * NCHT, CH)
    dstp = jnp.concatenate([ei[1], loop, pad]).reshape(NS * NCHT, CH)
    xp = jnp.zeros((NP, IN_DIM), jnp.float32).at[:N, :].set(x)
    zrow1 = jnp.zeros((RPT,), jnp.float32)
    ones_c = jnp.ones((CH,), jnp.float32)

    degp = _make_deg()(dstp, ones_c, zrow1)
    y1, dinv = _tc_layer1(xp, degp, W1)
    agg1 = _make_agg(HID // 2, True, NCHT, 6)(y1, srcp, dstp)
    y2 = _tc_mid(agg1, dinv, b1, W2)
    agg2 = _make_agg(OUT_DIM, False, NCHD, 3)(y2, srcp, dstp)
    outp = _tc_out(agg2, dinv, b2)
    return outp[:N]


# speedup vs baseline: 1.2018x; 1.2018x over previous
import functools

import jax
import jax.numpy as jnp
from jax import lax
from jax.experimental import pallas as pl
from jax.experimental.pallas import tpu as pltpu
from jax.experimental.pallas import tpu_sc as plsc

N = 10000
E = 320000
EL = E + N
IN_DIM = 128
HID = 64
OUT_DIM = 32

NC = 2
NS = 16
NW = NC * NS
CH = 128
NCHT = 162
EPAD = NS * NCHT * CH
NCHD = EPAD // (NW * CH)
NP = 10240
RPT = NP // NS
DUMP = N
K = 9
NG = NCHT // K

RB = 2048


def _sc_mesh():
    return plsc.VectorSubcoreMesh(
        core_axis_name="c", subcore_axis_name="s",
        num_cores=NC, num_subcores=NS)


@functools.lru_cache(maxsize=None)
def _make_agg(dh):
    @functools.partial(
        pl.kernel,
        out_type=pltpu.HBM((NC, NP, dh), jnp.float32),
        mesh=_sc_mesh(),
        compiler_params=pltpu.CompilerParams(use_tc_tiling_on_sc=False),
        scratch_types=(
            [pltpu.VMEM((NCHT, CH), jnp.int32)] * 2
            + [pltpu.VMEM((CH, dh), jnp.float32)] * (K + 1)
            + [pltpu.SemaphoreType.DMA] * (2 * K)
            + [pltpu.VMEM_SHARED((NP, dh), jnp.float32)] * 2
        ),
    )
    def agg_kernel(y_hbm, src_hbm, dst_hbm, out_hbm, src_idx, dst_idx, *rest):
        bufs = rest[:K]
        zb = rest[K]
        gsem = rest[K + 1:2 * K + 1]
        ssem = rest[2 * K + 1:3 * K + 1]
        acc = rest[3 * K + 1]
        y_sh = rest[3 * K + 2]
        c = lax.axis_index("c")
        s = lax.axis_index("s")
        pltpu.sync_copy(src_hbm.at[pl.ds(s * NCHT, NCHT)], src_idx)
        pltpu.sync_copy(dst_hbm.at[pl.ds(s * NCHT, NCHT)], dst_idx)
        nq = RPT // CH
        for q in range(2):
            pltpu.async_copy(y_hbm.at[c, pl.ds(s * RPT + q * CH, CH)],
                             bufs[q], gsem[q])
        for q in range(nq):
            pltpu.make_async_copy(y_hbm.at[c, pl.ds(s * RPT + q * CH, CH)],
                                  bufs[q % 2], gsem[q % 2]).wait()
            pltpu.sync_copy(bufs[q % 2], y_sh.at[pl.ds(s * RPT + q * CH, CH)])
            if q + 2 < nq:
                pltpu.async_copy(
                    y_hbm.at[c, pl.ds(s * RPT + (q + 2) * CH, CH)],
                    bufs[q % 2], gsem[q % 2])
        zeros16 = jnp.zeros((16,), jnp.float32)

        @pl.loop(0, CH)
        def zrow(i):
            for k in range(dh // 16):
                zb[i, pl.ds(k * 16, 16)] = zeros16

        for q in range(nq):
            pltpu.sync_copy(zb, acc.at[pl.ds(s * RPT + q * CH, CH)])
        plsc.subcore_barrier()
        for b in range(K):
            pltpu.async_copy(y_sh.at[src_idx.at[b]], bufs[b], gsem[b])

        @pl.loop(0, NG - 1)
        def group(g):
            base = g * K
            for b in range(K):
                pltpu.make_async_copy(
                    y_sh.at[src_idx.at[base + b]], bufs[b], gsem[b]).wait()
                pltpu.async_copy(bufs[b], acc.at[dst_idx.at[base + b]],
                                 ssem[b], add=True)
            for b in range(K):
                pltpu.make_async_copy(
                    bufs[b], acc.at[dst_idx.at[base + b]], ssem[b]).wait()
                pltpu.async_copy(
                    y_sh.at[src_idx.at[base + K + b]], bufs[b], gsem[b])

        base = (NG - 1) * K
        for b in range(K):
            pltpu.make_async_copy(
                y_sh.at[src_idx.at[base + b]], bufs[b], gsem[b]).wait()
            pltpu.async_copy(bufs[b], acc.at[dst_idx.at[base + b]],
                             ssem[b], add=True)
        for b in range(K):
            pltpu.make_async_copy(
                bufs[b], acc.at[dst_idx.at[base + b]], ssem[b]).wait()
        plsc.subcore_barrier()
        pltpu.sync_copy(acc.at[pl.ds(s * RPT, RPT)],
                        out_hbm.at[c, pl.ds(s * RPT, RPT)])

    return agg_kernel


@functools.lru_cache(maxsize=None)
def _make_deg():
    @functools.partial(
        pl.kernel,
        out_type=pltpu.HBM((NC, NP), jnp.float32),
        mesh=_sc_mesh(),
        compiler_params=pltpu.CompilerParams(use_tc_tiling_on_sc=False),
        scratch_types=(
            pltpu.VMEM((NCHD, CH), jnp.int32),
            pltpu.VMEM((CH,), jnp.float32),
            pltpu.VMEM_SHARED((NP,), jnp.float32),
        ),
    )
    def deg_kernel(dst_hbm, ones_hbm, zrow_hbm, out_hbm, dst_idx, ones_v, acc):
        c = lax.axis_index("c")
        s = lax.axis_index("s")
        wid = s * NC + c
        pltpu.sync_copy(zrow_hbm, acc.at[pl.ds(s * RPT, RPT)])
        pltpu.sync_copy(ones_hbm, ones_v)
        pltpu.sync_copy(dst_hbm.at[pl.ds(wid * NCHD, NCHD)], dst_idx)
        plsc.subcore_barrier()

        @pl.loop(0, NCHD)
        def chunk(j):
            pltpu.sync_copy(ones_v, acc.at[dst_idx.at[j]], add=True)

        plsc.subcore_barrier()
        pltpu.sync_copy(acc.at[pl.ds(s * RPT, RPT)],
                        out_hbm.at[c, pl.ds(s * RPT, RPT)])

    return deg_kernel


def _tc_layer1(xp, degp, W1):
    H2 = HID // 2

    def body(x_ref, deg_ref, w_ref, y_ref, dinv_ref):
        deg = deg_ref[0, :] + deg_ref[1, :]
        s = lax.rsqrt(deg)
        y = jnp.dot(x_ref[...], w_ref[...],
                    preferred_element_type=jnp.float32) * s[:, None]
        y_ref[0, :, :] = y[:, :H2]
        y_ref[1, :, :] = y[:, H2:]
        dinv_ref[0, 0, :] = s

    return pl.pallas_call(
        body,
        grid=(NP // RB,),
        in_specs=[
            pl.BlockSpec((RB, IN_DIM), lambda i: (i, 0)),
            pl.BlockSpec((NC, RB), lambda i: (0, i)),
            pl.BlockSpec((IN_DIM, HID), lambda i: (0, 0)),
        ],
        out_specs=[
            pl.BlockSpec((NC, RB, H2), lambda i: (0, i, 0)),
            pl.BlockSpec((1, 1, RB), lambda i: (i, 0, 0)),
        ],
        out_shape=[
            jax.ShapeDtypeStruct((NC, NP, H2), jnp.float32),
            jax.ShapeDtypeStruct((NP // RB, 1, RB), jnp.float32),
        ],
    )(xp, degp, W1)


def _tc_mid(agg1, dinv, b1, W2):
    H2 = HID // 2
    O2 = OUT_DIM // 2

    def body(a_ref, dinv_ref, b_ref, w_ref, y2_ref):
        s = dinv_ref[0, 0, :]
        b = b_ref[...]
        w = w_ref[...]
        hl = jnp.maximum(a_ref[0] * s[:, None] + b[:H2], 0.0)
        hr = jnp.maximum(a_ref[1] * s[:, None] + b[H2:], 0.0)
        y2 = (jnp.dot(hl, w[:H2, :], preferred_element_type=jnp.float32)
              + jnp.dot(hr, w[H2:, :], preferred_element_type=jnp.float32))
        y2 = y2 * s[:, None]
        y2_ref[0, :, :] = y2[:, :O2]
        y2_ref[1, :, :] = y2[:, O2:]

    return pl.pallas_call(
        body,
        grid=(NP // RB,),
        in_specs=[
            pl.BlockSpec((NC, RB, H2), lambda i: (0, i, 0)),
            pl.BlockSpec((1, 1, RB), lambda i: (i, 0, 0)),
            pl.BlockSpec((HID,), lambda i: (0,)),
            pl.BlockSpec((HID, OUT_DIM), lambda i: (0, 0)),
        ],
        out_specs=pl.BlockSpec((NC, RB, O2), lambda i: (0, i, 0)),
        out_shape=jax.ShapeDtypeStruct((NC, NP, O2), jnp.float32),
    )(agg1, dinv, b1, W2)


def _tc_out(agg2, dinv, b2):
    O2 = OUT_DIM // 2

    def body(a_ref, dinv_ref, b_ref, o_ref):
        s = dinv_ref[0, 0, :]
        o = jnp.concatenate([a_ref[0], a_ref[1]], axis=1) * s[:, None] + b_ref[...]
        o_ref[...] = jnp.maximum(o, 0.0)

    return pl.pallas_call(
        body,
        grid=(NP // RB,),
        in_specs=[
            pl.BlockSpec((NC, RB, O2), lambda i: (0, i, 0)),
            pl.BlockSpec((1, 1, RB), lambda i: (i, 0, 0)),
            pl.BlockSpec((OUT_DIM,), lambda i: (0,)),
        ],
        out_specs=pl.BlockSpec((RB, OUT_DIM), lambda i: (i, 0)),
        out_shape=jax.ShapeDtypeStruct((NP, OUT_DIM), jnp.float32),
    )(agg2, dinv, b2)


def kernel(x, edge_index, W1, b1, W2, b2):
    ei = edge_index.astype(jnp.int32)
    loop = jnp.arange(N, dtype=jnp.int32)
    pad = jnp.full((EPAD - EL,), DUMP, jnp.int32)
    srcp = jnp.concatenate([ei[0], loop, pad]).reshape(NS * NCHT, CH)
    dstp = jnp.concatenate([ei[1], loop, pad]).reshape(NS * NCHT, CH)
    xp = jnp.zeros((NP, IN_DIM), jnp.float32).at[:N, :].set(x)
    zrow1 = jnp.zeros((RPT,), jnp.float32)
    ones_c = jnp.ones((CH,), jnp.float32)

    degp = _make_deg()(dstp, ones_c, zrow1)
    y1, dinv = _tc_layer1(xp, degp, W1)
    agg1 = _make_agg(HID // 2)(y1, srcp, dstp)
    y2 = _tc_mid(agg1, dinv, b1, W2)
    agg2 = _make_agg(OUT_DIM // 2)(y2, srcp, dstp)
    outp = _tc_out(agg2, dinv, b2)
    return outp[:N]
